# Initial kernel scaffold; baseline (speedup 1.0000x reference)
import jax
import jax.numpy as jnp
from jax.experimental import pallas as pl


def kernel(rel_ids, emb_weight):
    raise NotImplementedError("write your pallas kernel here")


# speedup vs baseline: 5.5124x; 5.5124x over previous
import functools

import jax
import jax.numpy as jnp
from jax import lax
from jax.experimental import pallas as pl
from jax.experimental.pallas import tpu as pltpu
from jax.experimental.pallas import tpu_sc as plsc

_D = 64
_L = 128
_K = 5
_C = _K * _L
_NBUF = 2


def _emb_body(nchunk, ids_hbm, table_hbm, out_hbm,
              idx_v, rows_v, gsem0, gsem1, wsem0, wsem1, isem):
    nc = 2
    wid = lax.axis_index("s") * nc + lax.axis_index("c")
    idx0 = wid * (nchunk * _C)
    out_row0 = wid * (nchunk * _C)
    gsems = (gsem0, gsem1)
    wsems = (wsem0, wsem1)

    def load_and_fire(b, g):
        handles = [
            pltpu.async_copy(
                ids_hbm.at[pl.ds(idx0 + g * _C + j * _L, _L)],
                idx_v.at[b, j], isem)
            for j in range(_K)
        ]
        for h in handles:
            h.wait()
        for j in range(_K):
            pltpu.async_copy(
                table_hbm.at[idx_v.at[b, j]],
                rows_v.at[b, pl.ds(j * _L, _L), :],
                gsems[b])

    for b in range(_NBUF):
        load_and_fire(b, b)

    @pl.loop(0, nchunk, step=_NBUF)
    def _chunks(g0):
        for b in range(_NBUF):
            g = g0 + b
            pltpu.make_async_copy(
                table_hbm.at[pl.ds(0, _C), :], rows_v.at[b],
                gsems[b]).wait()
            wcopy = pltpu.async_copy(
                rows_v.at[b],
                out_hbm.at[pl.ds(out_row0 + g * _C, _C), :],
                wsems[b])
            wcopy.wait()

            @pl.when(g + _NBUF < nchunk)
            def _refill():
                load_and_fire(b, g + _NBUF)


def kernel(rel_ids, emb_weight):
    e = rel_ids.size
    nw = 32
    bpw = e // nw
    nchunk = bpw // _C
    assert bpw % _C == 0 and e % (nw * _L) == 0

    ids1d = rel_ids.reshape(-1).astype(jnp.int32)
    mesh = plsc.VectorSubcoreMesh(core_axis_name="c", subcore_axis_name="s")

    run = pl.kernel(
        functools.partial(_emb_body, nchunk),
        out_type=jax.ShapeDtypeStruct((e, _D), jnp.float32),
        mesh=mesh,
        scratch_types=[
            pltpu.VMEM((_NBUF, _K, _L), jnp.int32),
            pltpu.VMEM((_NBUF, _C, _D), jnp.float32),
            pltpu.SemaphoreType.DMA,
            pltpu.SemaphoreType.DMA,
            pltpu.SemaphoreType.DMA,
            pltpu.SemaphoreType.DMA,
            pltpu.SemaphoreType.DMA,
        ],
        compiler_params=pltpu.CompilerParams(use_tc_tiling_on_sc=False),
    )
    return run(ids1d, emb_weight)


# speedup vs baseline: 5.5876x; 1.0136x over previous
import functools

import jax
import jax.numpy as jnp
from jax import lax
from jax.experimental import pallas as pl
from jax.experimental.pallas import tpu as pltpu
from jax.experimental.pallas import tpu_sc as plsc

_D = 64
_L = 128
_K = 2
_C = _K * _L
_NBUF = 4
_LA = 2


def _emb_body(nchunk, ids_hbm, table_hbm, out_hbm, idx_v, rows_v,
              gsem0, gsem1, gsem2, gsem3, wsem0, wsem1, wsem2, wsem3):
    nc = 2
    wid = lax.axis_index("s") * nc + lax.axis_index("c")
    base = wid * (nchunk * _C)
    gsems = (gsem0, gsem1, gsem2, gsem3)
    wsems = (wsem0, wsem1, wsem2, wsem3)

    def load_and_fire(b, g):
        pltpu.sync_copy(ids_hbm.at[pl.ds(base + g * _C, _C)], idx_v.at[b])
        for j in range(_K):
            pltpu.async_copy(
                table_hbm.at[idx_v.at[b, pl.ds(j * _L, _L)]],
                rows_v.at[b, pl.ds(j * _L, _L), :],
                gsems[b])

    def drain_gather(b):
        pltpu.make_async_copy(
            table_hbm.at[pl.ds(0, _C), :], rows_v.at[b], gsems[b]).wait()

    def drain_write(b):
        pltpu.make_async_copy(
            table_hbm.at[pl.ds(0, _C), :], rows_v.at[b], wsems[b]).wait()

    for g in range(_LA):
        load_and_fire(g % _NBUF, g)

    @pl.loop(0, nchunk, step=_NBUF)
    def _chunks(g0):
        for b in range(_NBUF):
            g = g0 + b
            drain_gather(b)
            pltpu.async_copy(
                rows_v.at[b],
                out_hbm.at[pl.ds(base + g * _C, _C), :],
                wsems[b])
            h = g + _LA
            b2 = (b + _LA) % _NBUF

            @pl.when(h < nchunk)
            def _refill():
                @pl.when(h >= _NBUF)
                def _free_buf():
                    drain_write(b2)

                load_and_fire(b2, h)

    for b in range(_NBUF):
        drain_write(b)


def kernel(rel_ids, emb_weight):
    e = rel_ids.size
    nw = 32
    bpw = e // nw
    nchunk = bpw // _C
    assert bpw % (_C * _NBUF) == 0

    ids1d = rel_ids.reshape(-1).astype(jnp.int32)
    mesh = plsc.VectorSubcoreMesh(core_axis_name="c", subcore_axis_name="s")

    run = pl.kernel(
        functools.partial(_emb_body, nchunk),
        out_type=jax.ShapeDtypeStruct((e, _D), jnp.float32),
        mesh=mesh,
        scratch_types=[
            pltpu.VMEM((_NBUF, _C), jnp.int32),
            pltpu.VMEM((_NBUF, _C, _D), jnp.float32),
            pltpu.SemaphoreType.DMA,
            pltpu.SemaphoreType.DMA,
            pltpu.SemaphoreType.DMA,
            pltpu.SemaphoreType.DMA,
            pltpu.SemaphoreType.DMA,
            pltpu.SemaphoreType.DMA,
            pltpu.SemaphoreType.DMA,
            pltpu.SemaphoreType.DMA,
        ],
        compiler_params=pltpu.CompilerParams(use_tc_tiling_on_sc=False),
    )
    return run(ids1d, emb_weight)


# speedup vs baseline: 8.2102x; 1.4694x over previous
import functools

import jax
import jax.numpy as jnp
from jax import lax
from jax.experimental import pallas as pl
from jax.experimental.pallas import tpu as pltpu
from jax.experimental.pallas import tpu_sc as plsc

_D = 64
_L = 128
_K = 2
_C = _K * _L
_NBUF = 4
_LA = 2


def _emb_body(nchunk, ids_hbm, table_hbm, out_hbm, idx_v, rows_v,
              gsem0, gsem1, gsem2, gsem3, wsem0, wsem1, wsem2, wsem3):
    wid = lax.axis_index("s") * 2 + lax.axis_index("c")
    bpw = nchunk * _C
    base = wid * bpw
    gsems = (gsem0, gsem1, gsem2, gsem3)
    wsems = (wsem0, wsem1, wsem2, wsem3)

    def out_slice(g):
        t0 = base + g * _C
        iblk = t0 // (2 * _BLK)
        u = t0 % (2 * _BLK)
        h = u // _BLK
        q = u % _BLK
        return pl.ds(iblk * _BLK + q, _C), pl.ds(h * _D, _D)

    def load_and_fire(b, g):
        pltpu.sync_copy(ids_hbm.at[pl.ds(base + g * _C, _C)], idx_v.at[b])
        for j in range(_K):
            pltpu.async_copy(
                table_hbm.at[idx_v.at[b, pl.ds(j * _L, _L)]],
                rows_v.at[b, pl.ds(j * _L, _L), :],
                gsems[b])

    def drain_gather(b):
        pltpu.make_async_copy(
            table_hbm.at[pl.ds(0, _C), :], rows_v.at[b], gsems[b]).wait()

    def drain_write(b):
        pltpu.make_async_copy(
            table_hbm.at[pl.ds(0, _C), :], rows_v.at[b], wsems[b]).wait()

    for g in range(_LA):
        load_and_fire(g % _NBUF, g)

    @pl.loop(0, nchunk, step=_NBUF)
    def _chunks(g0):
        for b in range(_NBUF):
            g = g0 + b
            drain_gather(b)
            rs, cs = out_slice(g)
            pltpu.async_copy(rows_v.at[b], out_hbm.at[rs, cs], wsems[b])
            h = g + _LA
            b2 = (b + _LA) % _NBUF

            @pl.when(h < nchunk)
            def _refill():
                @pl.when(h >= _NBUF)
                def _free_buf():
                    drain_write(b2)

                load_and_fire(b2, h)

    for b in range(_NBUF):
        drain_write(b)


_BLK = 2048


def _transpose_body(x_ref, o_ref):
    x = x_ref[...]
    o_ref[:, :_BLK] = x[:, :_D].T
    o_ref[:, _BLK:] = x[:, _D:].T


def _tc_transpose(y2, e):
    nb = e // (2 * _BLK)
    return pl.pallas_call(
        _transpose_body,
        grid=(nb,),
        in_specs=[pl.BlockSpec((_BLK, 2 * _D), lambda i: (i, 0))],
        out_specs=pl.BlockSpec((_D, 2 * _BLK), lambda i: (0, i)),
        out_shape=jax.ShapeDtypeStruct((_D, e), jnp.float32),
    )(y2)


def kernel(rel_ids, emb_weight):
    e = rel_ids.size
    nw = 32
    bpw = e // nw
    nchunk = bpw // _C
    assert bpw % (_C * _NBUF) == 0

    ids1d = rel_ids.reshape(-1).astype(jnp.int32)
    mesh = plsc.VectorSubcoreMesh(core_axis_name="c", subcore_axis_name="s")

    run = pl.kernel(
        functools.partial(_emb_body, nchunk),
        out_type=jax.ShapeDtypeStruct((e // 2, 2 * _D), jnp.float32),
        mesh=mesh,
        scratch_types=[
            pltpu.VMEM((_NBUF, _C), jnp.int32),
            pltpu.VMEM((_NBUF, _C, _D), jnp.float32),
            pltpu.SemaphoreType.DMA,
            pltpu.SemaphoreType.DMA,
            pltpu.SemaphoreType.DMA,
            pltpu.SemaphoreType.DMA,
            pltpu.SemaphoreType.DMA,
            pltpu.SemaphoreType.DMA,
            pltpu.SemaphoreType.DMA,
            pltpu.SemaphoreType.DMA,
        ],
        compiler_params=pltpu.CompilerParams(use_tc_tiling_on_sc=False),
    )
    y2 = run(ids1d, emb_weight)
    return _tc_transpose(y2, e).T


# speedup vs baseline: 10.4352x; 1.2710x over previous
import functools

import jax
import jax.numpy as jnp
from jax import lax
from jax.experimental import pallas as pl
from jax.experimental.pallas import tpu as pltpu
from jax.experimental.pallas import tpu_sc as plsc

_D = 64
_L = 128
_K = 2
_C = _K * _L
_NBUF = 4
_LA = 2


def _emb_body(nchunk, ids_hbm, table_hbm, out_hbm, idx_v, rows_v,
              gsem0, gsem1, gsem2, gsem3, wsem0, wsem1, wsem2, wsem3):
    wid = lax.axis_index("s") * 2 + lax.axis_index("c")
    bpw = nchunk * _C
    base = wid * bpw
    gsems = (gsem0, gsem1, gsem2, gsem3)
    wsems = (wsem0, wsem1, wsem2, wsem3)

    def out_slice(g):
        t0 = base + g * _C
        iblk = t0 // (2 * _BLK)
        u = t0 % (2 * _BLK)
        h = u // _BLK
        q = u % _BLK
        return pl.ds(iblk * _BLK + q, _C), pl.ds(h * _D, _D)

    def load_and_fire(b, g):
        pltpu.sync_copy(ids_hbm.at[pl.ds(base + g * _C, _C)], idx_v.at[b])
        for j in range(_K):
            pltpu.async_copy(
                table_hbm.at[idx_v.at[b, pl.ds(j * _L, _L)]],
                rows_v.at[b, pl.ds(j * _L, _L), :],
                gsems[b])

    def drain_gather(b):
        pltpu.make_async_copy(
            table_hbm.at[pl.ds(0, _C), :], rows_v.at[b], gsems[b]).wait()

    def drain_write(b):
        pltpu.make_async_copy(
            table_hbm.at[pl.ds(0, _C), :], rows_v.at[b], wsems[b]).wait()

    for g in range(_LA):
        load_and_fire(g % _NBUF, g)

    @pl.loop(0, nchunk, step=_NBUF)
    def _chunks(g0):
        for b in range(_NBUF):
            g = g0 + b
            drain_gather(b)
            rs, cs = out_slice(g)
            pltpu.async_copy(rows_v.at[b], out_hbm.at[rs, cs], wsems[b])
            h = g + _LA
            b2 = (b + _LA) % _NBUF

            @pl.when(h < nchunk)
            def _refill():
                @pl.when(h >= _NBUF)
                def _free_buf():
                    drain_write(b2)

                load_and_fire(b2, h)

    for b in range(_NBUF):
        drain_write(b)


_BLK = 8192


def _transpose_body(x_ref, o_ref):
    x = x_ref[...]
    o_ref[:, :_BLK] = x[:, :_D].T
    o_ref[:, _BLK:] = x[:, _D:].T


def _tc_transpose(y2, e):
    nb = e // (2 * _BLK)
    return pl.pallas_call(
        _transpose_body,
        grid=(nb,),
        in_specs=[pl.BlockSpec((_BLK, 2 * _D), lambda i: (i, 0))],
        out_specs=pl.BlockSpec((_D, 2 * _BLK), lambda i: (0, i)),
        out_shape=jax.ShapeDtypeStruct((_D, e), jnp.float32),
    )(y2)


def kernel(rel_ids, emb_weight):
    e = rel_ids.size
    nw = 32
    bpw = e // nw
    nchunk = bpw // _C
    assert bpw % (_C * _NBUF) == 0

    ids1d = rel_ids.reshape(-1).astype(jnp.int32)
    mesh = plsc.VectorSubcoreMesh(core_axis_name="c", subcore_axis_name="s")

    run = pl.kernel(
        functools.partial(_emb_body, nchunk),
        out_type=jax.ShapeDtypeStruct((e // 2, 2 * _D), jnp.float32),
        mesh=mesh,
        scratch_types=[
            pltpu.VMEM((_NBUF, _C), jnp.int32),
            pltpu.VMEM((_NBUF, _C, _D), jnp.float32),
            pltpu.SemaphoreType.DMA,
            pltpu.SemaphoreType.DMA,
            pltpu.SemaphoreType.DMA,
            pltpu.SemaphoreType.DMA,
            pltpu.SemaphoreType.DMA,
            pltpu.SemaphoreType.DMA,
            pltpu.SemaphoreType.DMA,
            pltpu.SemaphoreType.DMA,
        ],
        compiler_params=pltpu.CompilerParams(use_tc_tiling_on_sc=False),
    )
    y2 = run(ids1d, emb_weight)
    return _tc_transpose(y2, e).T


# speedup vs baseline: 10.8759x; 1.0422x over previous
import functools

import jax
import jax.numpy as jnp
from jax import lax
from jax.experimental import pallas as pl
from jax.experimental.pallas import tpu as pltpu
from jax.experimental.pallas import tpu_sc as plsc

_D = 64
_L = 128
_K = 2
_C = _K * _L
_NBUF = 4
_LA = 2


def _emb_body(nchunk, ids_hbm, table_hbm, out_hbm, idx_v, rows_v,
              gsem0, gsem1, gsem2, gsem3, wsem0, wsem1, wsem2, wsem3):
    wid = lax.axis_index("s") * 2 + lax.axis_index("c")
    bpw = nchunk * _C
    base = wid * bpw
    gsems = (gsem0, gsem1, gsem2, gsem3)
    wsems = (wsem0, wsem1, wsem2, wsem3)

    def out_slice(g):
        t0 = base + g * _C
        iblk = t0 // (2 * _BLK)
        u = t0 % (2 * _BLK)
        h = u // _BLK
        q = u % _BLK
        return pl.ds(iblk * _BLK + q, _C), pl.ds(h * _D, _D)

    def load_and_fire(b, g):
        pltpu.sync_copy(ids_hbm.at[pl.ds(base + g * _C, _C)], idx_v.at[b])
        for j in range(_K):
            pltpu.async_copy(
                table_hbm.at[idx_v.at[b, pl.ds(j * _L, _L)]],
                rows_v.at[b, pl.ds(j * _L, _L), :],
                gsems[b])

    def drain_gather(b):
        pltpu.make_async_copy(
            table_hbm.at[pl.ds(0, _C), :], rows_v.at[b], gsems[b]).wait()

    def drain_write(b):
        pltpu.make_async_copy(
            table_hbm.at[pl.ds(0, _C), :], rows_v.at[b], wsems[b]).wait()

    for g in range(_LA):
        load_and_fire(g % _NBUF, g)

    @pl.loop(0, nchunk, step=_NBUF)
    def _chunks(g0):
        for b in range(_NBUF):
            g = g0 + b
            drain_gather(b)
            rs, cs = out_slice(g)
            pltpu.async_copy(rows_v.at[b], out_hbm.at[rs, cs], wsems[b])
            h = g + _LA
            b2 = (b + _LA) % _NBUF

            @pl.when(h < nchunk)
            def _refill():
                @pl.when(h >= _NBUF)
                def _free_buf():
                    drain_write(b2)

                load_and_fire(b2, h)

    for b in range(_NBUF):
        drain_write(b)


_BLK = 16384


def _transpose_body(x_ref, o_ref):
    x = x_ref[...]
    o_ref[:, :_BLK] = x[:, :_D].T
    o_ref[:, _BLK:] = x[:, _D:].T


def _tc_transpose(y2, e):
    nb = e // (2 * _BLK)
    return pl.pallas_call(
        _transpose_body,
        grid=(nb,),
        in_specs=[pl.BlockSpec((_BLK, 2 * _D), lambda i: (i, 0))],
        out_specs=pl.BlockSpec((_D, 2 * _BLK), lambda i: (0, i)),
        out_shape=jax.ShapeDtypeStruct((_D, e), jnp.float32),
    )(y2)


def kernel(rel_ids, emb_weight):
    e = rel_ids.size
    nw = 32
    bpw = e // nw
    nchunk = bpw // _C
    assert bpw % (_C * _NBUF) == 0

    ids1d = rel_ids.reshape(-1).astype(jnp.int32)
    mesh = plsc.VectorSubcoreMesh(core_axis_name="c", subcore_axis_name="s")

    run = pl.kernel(
        functools.partial(_emb_body, nchunk),
        out_type=jax.ShapeDtypeStruct((e // 2, 2 * _D), jnp.float32),
        mesh=mesh,
        scratch_types=[
            pltpu.VMEM((_NBUF, _C), jnp.int32),
            pltpu.VMEM((_NBUF, _C, _D), jnp.float32),
            pltpu.SemaphoreType.DMA,
            pltpu.SemaphoreType.DMA,
            pltpu.SemaphoreType.DMA,
            pltpu.SemaphoreType.DMA,
            pltpu.SemaphoreType.DMA,
            pltpu.SemaphoreType.DMA,
            pltpu.SemaphoreType.DMA,
            pltpu.SemaphoreType.DMA,
        ],
        compiler_params=pltpu.CompilerParams(use_tc_tiling_on_sc=False),
    )
    y2 = run(ids1d, emb_weight)
    return _tc_transpose(y2, e).T
